# baseline (speedup 1.0000x reference)
import functools

import jax
import jax.numpy as jnp
from jax import lax
from jax.experimental import pallas as pl
from jax.experimental.pallas import tpu as pltpu
from jax.experimental.pallas import tpu_sc as plsc

_N_TOKENS = 16384
_NUM_CODES = 1024
_DIM = 64
_TILE = 1024
_GRID = _N_TOKENS // _TILE

_NC = 2
_NS = 16
_NW = _NC * _NS
_CHUNK = 128
_B_PER_W = _N_TOKENS // _NW
_N_CHUNKS = _B_PER_W // _CHUNK


def _vq_tc_body(z_ref, e_ref, idx_ref, loss_ref):
    z = z_ref[...]
    e = e_ref[...]
    zsq = jnp.sum(z * z, axis=1, keepdims=True)
    esq = jnp.sum(e * e, axis=1)[None, :]
    mm = lax.dot_general(
        z, e, (((1,), (1,)), ((), ())), preferred_element_type=jnp.float32
    )
    d = (zsq - 2.0 * mm) + esq
    mins = jnp.min(d, axis=1, keepdims=True)
    ii = lax.broadcasted_iota(jnp.int32, (_TILE, _NUM_CODES), 1)
    idx = jnp.min(jnp.where(d == mins, ii, _NUM_CODES), axis=1)
    idx_ref[...] = idx
    tile_sum = jnp.sum(mins)

    @pl.when(pl.program_id(0) == 0)
    def _():
        loss_ref[0, 0] = 0.0

    loss_ref[0, 0] += tile_sum


def _tc_argmin(z, embeddings):
    return pl.pallas_call(
        _vq_tc_body,
        grid=(_GRID,),
        in_specs=[
            pl.BlockSpec((_TILE, _DIM), lambda i: (i, 0)),
            pl.BlockSpec((_NUM_CODES, _DIM), lambda i: (0, 0)),
        ],
        out_specs=(
            pl.BlockSpec((_TILE,), lambda i: (i,)),
            pl.BlockSpec(memory_space=pltpu.SMEM),
        ),
        out_shape=(
            jax.ShapeDtypeStruct((_N_TOKENS,), jnp.int32),
            jax.ShapeDtypeStruct((1, 1), jnp.float32),
        ),
        compiler_params=pltpu.CompilerParams(
            dimension_semantics=("arbitrary",),
        ),
    )(z, embeddings)


@functools.partial(
    pl.kernel,
    mesh=plsc.VectorSubcoreMesh(core_axis_name="c", subcore_axis_name="s"),
    out_type=jax.ShapeDtypeStruct((_N_TOKENS, _DIM), jnp.float32),
    scratch_types=[
        pltpu.VMEM((_N_CHUNKS, _CHUNK), jnp.int32),
        pltpu.VMEM((_N_CHUNKS, _CHUNK, _DIM), jnp.float32),
        pltpu.SemaphoreType.DMA,
    ],
    compiler_params=pltpu.CompilerParams(use_tc_tiling_on_sc=False),
)
def _sc_gather(table_hbm, idx_hbm, out_hbm, idx_v, rows_v, sem):
    wid = lax.axis_index("s") * _NC + lax.axis_index("c")
    base = wid * _B_PER_W
    for c in range(_N_CHUNKS):
        pltpu.sync_copy(idx_hbm.at[pl.ds(base + c * _CHUNK, _CHUNK)], idx_v.at[c])
    copies = [
        pltpu.async_copy(table_hbm.at[idx_v.at[c]], rows_v.at[c], sem)
        for c in range(_N_CHUNKS)
    ]
    for cp in copies:
        cp.wait()
    for c in range(_N_CHUNKS):
        pltpu.sync_copy(rows_v.at[c], out_hbm.at[pl.ds(base + c * _CHUNK, _CHUNK)])


def kernel(z, embeddings):
    idx, loss_acc = _tc_argmin(z, embeddings)
    q = _sc_gather(embeddings, idx)
    loss = loss_acc[0, 0] / (_N_TOKENS * _DIM)
    return q, idx, loss


# speedup vs baseline: 2.6406x; 2.6406x over previous
import jax
import jax.numpy as jnp
from jax import lax
from jax.experimental import pallas as pl
from jax.experimental.pallas import tpu as pltpu

_N_TOKENS = 16384
_NUM_CODES = 1024
_DIM = 64
_TILE = 1024
_GRID = _N_TOKENS // _TILE


def _vq_body(zt_ref, et_ref, qt_ref, idx_ref, loss_ref):
    zt = zt_ref[...]
    et = et_ref[...]
    zsq = jnp.sum(zt * zt, axis=0, keepdims=True)
    esq = jnp.sum(et * et, axis=0)[:, None]
    mmt = lax.dot_general(
        et, zt, (((0,), (0,)), ((), ())), preferred_element_type=jnp.float32
    )
    d = (zsq - 2.0 * mmt) + esq
    mins = jnp.min(d, axis=0, keepdims=True)
    ii = lax.broadcasted_iota(jnp.int32, (_NUM_CODES, _TILE), 0)
    idx = jnp.min(jnp.where(d == mins, ii, _NUM_CODES), axis=0)
    idx_ref[...] = idx
    onehot = (ii == idx[None, :]).astype(jnp.float32)
    qt_ref[...] = lax.dot_general(
        et, onehot, (((1,), (0,)), ((), ())), preferred_element_type=jnp.float32
    )
    tile_sum = jnp.sum(mins)

    @pl.when(pl.program_id(0) == 0)
    def _():
        loss_ref[0, 0] = 0.0

    loss_ref[0, 0] += tile_sum


def kernel(z, embeddings):
    zt = z.T
    et = embeddings.T
    qt, idx, loss_acc = pl.pallas_call(
        _vq_body,
        grid=(_GRID,),
        in_specs=[
            pl.BlockSpec((_DIM, _TILE), lambda i: (0, i)),
            pl.BlockSpec((_DIM, _NUM_CODES), lambda i: (0, 0)),
        ],
        out_specs=(
            pl.BlockSpec((_DIM, _TILE), lambda i: (0, i)),
            pl.BlockSpec((_TILE,), lambda i: (i,)),
            pl.BlockSpec(memory_space=pltpu.SMEM),
        ),
        out_shape=(
            jax.ShapeDtypeStruct((_DIM, _N_TOKENS), jnp.float32),
            jax.ShapeDtypeStruct((_N_TOKENS,), jnp.int32),
            jax.ShapeDtypeStruct((1, 1), jnp.float32),
        ),
        compiler_params=pltpu.CompilerParams(
            dimension_semantics=("arbitrary",),
        ),
    )(zt, et)
    loss = loss_acc[0, 0] / (_N_TOKENS * _DIM)
    return qt.T, idx, loss


# speedup vs baseline: 3.0778x; 1.1656x over previous
import jax
import jax.numpy as jnp
from jax import lax
from jax.experimental import pallas as pl
from jax.experimental.pallas import tpu as pltpu

_N_TOKENS = 16384
_NUM_CODES = 1024
_DIM = 64
_TILE = 4096
_GRID = _N_TOKENS // _TILE


def _vq_body(zt_ref, et_ref, qt_ref, idx_ref, loss_ref):
    zt = zt_ref[...]
    et = et_ref[...]
    zsq = jnp.sum(zt * zt, axis=0, keepdims=True)
    esq = jnp.sum(et * et, axis=0)[:, None]
    mmt = lax.dot_general(
        et, zt, (((0,), (0,)), ((), ())), preferred_element_type=jnp.float32
    )
    d = (zsq - 2.0 * mmt) + esq
    mins = jnp.min(d, axis=0, keepdims=True)
    ii = lax.broadcasted_iota(jnp.int32, (_NUM_CODES, _TILE), 0)
    idx = jnp.min(jnp.where(d == mins, ii, _NUM_CODES), axis=0)
    idx_ref[...] = idx
    onehot = (ii == idx[None, :]).astype(jnp.float32)
    qt_ref[...] = lax.dot_general(
        et, onehot, (((1,), (0,)), ((), ())), preferred_element_type=jnp.float32
    )
    tile_sum = jnp.sum(mins)

    @pl.when(pl.program_id(0) == 0)
    def _():
        loss_ref[0, 0] = 0.0

    loss_ref[0, 0] += tile_sum


def kernel(z, embeddings):
    zt = z.T
    et = embeddings.T
    qt, idx, loss_acc = pl.pallas_call(
        _vq_body,
        grid=(_GRID,),
        in_specs=[
            pl.BlockSpec((_DIM, _TILE), lambda i: (0, i)),
            pl.BlockSpec((_DIM, _NUM_CODES), lambda i: (0, 0)),
        ],
        out_specs=(
            pl.BlockSpec((_DIM, _TILE), lambda i: (0, i)),
            pl.BlockSpec((_TILE,), lambda i: (i,)),
            pl.BlockSpec(memory_space=pltpu.SMEM),
        ),
        out_shape=(
            jax.ShapeDtypeStruct((_DIM, _N_TOKENS), jnp.float32),
            jax.ShapeDtypeStruct((_N_TOKENS,), jnp.int32),
            jax.ShapeDtypeStruct((1, 1), jnp.float32),
        ),
        compiler_params=pltpu.CompilerParams(
            dimension_semantics=("arbitrary",),
        ),
    )(zt, et)
    loss = loss_acc[0, 0] / (_N_TOKENS * _DIM)
    return qt.T, idx, loss
